# baseline (speedup 1.0000x reference)
import jax
import jax.numpy as jnp
from jax.experimental import pallas as pl
from jax.experimental.pallas import tpu as pltpu

E = 160000
D = 256
N_DST = 10000
BM = 800


def _mm_kernel(x_ref, w_ref, b_ref, o_ref):
    x = x_ref[...]
    w = w_ref[...]
    acc = jax.lax.dot_general(x, w, (((1,), (1,)), ((), ())),
                              preferred_element_type=jnp.float32)
    o_ref[...] = jnp.maximum(acc + b_ref[...], 0.0)


def _edge_linear(src_emb, W, b):
    grid = (E // BM,)
    return pl.pallas_call(
        _mm_kernel,
        grid=grid,
        in_specs=[
            pl.BlockSpec((BM, D), lambda i: (i, 0)),
            pl.BlockSpec((D, D), lambda i: (0, 0)),
            pl.BlockSpec((1, D), lambda i: (0, 0)),
        ],
        out_specs=pl.BlockSpec((BM, D), lambda i: (i, 0)),
        out_shape=jax.ShapeDtypeStruct((E, D), jnp.float32),
    )(src_emb, W, b.reshape(1, D))


def kernel(block, src_emb, src_emb_in, W, b):
    msg = _edge_linear(src_emb, W, b)
    h = jax.ops.segment_max(msg, block, num_segments=N_DST)
    h = jnp.where(jnp.isneginf(h), 0.0, h)
    return h + src_emb[E:, :]
